# baseline (speedup 1.0000x reference)
import functools

import jax
import jax.numpy as jnp
from jax import lax
from jax.experimental import pallas as pl
from jax.experimental.pallas import tpu as pltpu
from jax.experimental.pallas import tpu_sc as plsc

SEQ = 2048
D_MODEL = 768
D_FF = 1024
E = 64
T = 64
NT = SEQ // T + E
NSLOT = NT * T


_TLOG = T.bit_length() - 1
assert (1 << _TLOG) == T


def _gate_body(x_ref, w_ref, b_ref, slot_ref, te_ref, tv_ref):
    scores = jnp.dot(x_ref[...], w_ref[...],
                     preferred_element_type=jnp.float32) + b_ref[...]
    m = jnp.max(scores, axis=1, keepdims=True)
    ii = lax.broadcasted_iota(jnp.int32, (SEQ, E), 1)
    eidx = jnp.min(jnp.where(scores == m, ii, E), axis=1, keepdims=True)
    onehot = (ii == eidx).astype(jnp.bfloat16)
    CKN, CK = 8, SEQ // 8
    rr = lax.broadcasted_iota(jnp.int32, (CK, CK), 0)
    cc = lax.broadcasted_iota(jnp.int32, (CK, CK), 1)
    ltri = (cc <= rr).astype(jnp.bfloat16)
    parts = [jnp.dot(ltri, onehot[k * CK:(k + 1) * CK, :],
                     preferred_element_type=jnp.float32) for k in range(CKN)]
    off = jnp.zeros((1, E), jnp.float32)
    fixed = []
    for k in range(CKN):
        fixed.append(parts[k] + off)
        off = off + parts[k][CK - 1:CK, :]
    csum = jnp.concatenate(fixed, axis=0)
    counts = csum[SEQ - 1:SEQ, :].astype(jnp.int32)
    onef = onehot.astype(jnp.float32)
    rank = (jnp.sum(onef * csum, axis=1, keepdims=True)
            - 1.0).astype(jnp.int32)
    tpe = (counts + (T - 1)) >> _TLOG
    ur = lax.broadcasted_iota(jnp.int32, (E, E), 0)
    uc = lax.broadcasted_iota(jnp.int32, (E, E), 1)
    utri = (ur <= uc).astype(jnp.bfloat16)
    ctiles = jnp.dot(tpe.astype(jnp.bfloat16), utri,
                     preferred_element_type=jnp.float32).astype(jnp.int32)
    tile_base = ctiles - tpe
    base_tok = jnp.sum(onef * tile_base.astype(jnp.float32),
                       axis=1, keepdims=True).astype(jnp.int32)
    slot_ref[...] = (base_tok + (rank >> _TLOG)) * T + (rank & (T - 1))
    jj = lax.broadcasted_iota(jnp.int32, (NT, E), 0)
    te = jnp.sum(jnp.where(ctiles <= jj, 1, 0), axis=1, keepdims=True)
    te_ref[...] = jnp.minimum(te, E - 1)
    total = ctiles[:, E - 1:E]
    jv = lax.broadcasted_iota(jnp.int32, (NT, 1), 0)
    tv_ref[...] = jnp.where(jv < total, 1, 0)


def _gate(x_flat, gate_W, gate_b):
    slot2, te2, tv2 = pl.pallas_call(
        _gate_body,
        out_shape=(
            jax.ShapeDtypeStruct((SEQ, 1), jnp.int32),
            jax.ShapeDtypeStruct((NT, 1), jnp.int32),
            jax.ShapeDtypeStruct((NT, 1), jnp.int32),
        ),
    )(x_flat, gate_W, gate_b.reshape(1, E))
    return slot2.reshape(SEQ), te2.reshape(NT), tv2.reshape(NT)


_NC, _NS = 2, 16
_NW = _NC * _NS


def _make_scatter_rows(n_rows, n_out, d):
    per_w = n_rows // _NW
    mesh = plsc.VectorSubcoreMesh(core_axis_name="c", subcore_axis_name="s")

    @functools.partial(
        pl.kernel, mesh=mesh,
        out_type=jax.ShapeDtypeStruct((n_out, d), jnp.float32),
        scratch_types=[
            pltpu.VMEM((per_w,), jnp.int32),
            pltpu.VMEM((per_w, d), jnp.float32),
            pltpu.SemaphoreType.DMA,
        ],
    )
    def k(table_hbm, idx_hbm, out_hbm, idx_v, rows_v, sem):
        wid = lax.axis_index("s") * _NC + lax.axis_index("c")
        base = wid * per_w
        pltpu.sync_copy(idx_hbm.at[pl.ds(base, per_w)], idx_v)
        pltpu.sync_copy(table_hbm.at[pl.ds(base, per_w)], rows_v)
        pltpu.async_copy(rows_v, out_hbm.at[idx_v], sem).wait()

    return k


def _make_gather_rows(n_rows, n_tab, d):
    per_w = n_rows // _NW
    mesh = plsc.VectorSubcoreMesh(core_axis_name="c", subcore_axis_name="s")

    @functools.partial(
        pl.kernel, mesh=mesh,
        out_type=jax.ShapeDtypeStruct((n_rows, d), jnp.float32),
        scratch_types=[
            pltpu.VMEM((per_w,), jnp.int32),
            pltpu.VMEM((per_w, d), jnp.float32),
            pltpu.SemaphoreType.DMA,
        ],
    )
    def k(table_hbm, idx_hbm, out_hbm, idx_v, rows_v, sem):
        wid = lax.axis_index("s") * _NC + lax.axis_index("c")
        base = wid * per_w
        pltpu.sync_copy(idx_hbm.at[pl.ds(base, per_w)], idx_v)
        pltpu.async_copy(table_hbm.at[idx_v], rows_v, sem).wait()
        pltpu.sync_copy(rows_v, out_hbm.at[pl.ds(base, per_w)])

    return k


_KH1 = D_MODEL // 2
_KH2 = D_FF // 2


def _ffn_body(te_ref, tv_ref, x_ref, w1a_ref, w1b_ref, b1_ref,
              w2a_ref, w2b_ref, b2_ref, o_ref):
    j = pl.program_id(0)

    @pl.when(tv_ref[j] == 1)
    def _():
        xt = x_ref[...]
        h = jnp.maximum(
            jnp.dot(xt[:, :_KH1], w1a_ref[0],
                    preferred_element_type=jnp.float32)
            + jnp.dot(xt[:, _KH1:], w1b_ref[0],
                      preferred_element_type=jnp.float32)
            + b1_ref[0], 0.0)
        o_ref[...] = (
            jnp.dot(h[:, :_KH2], w2a_ref[0],
                    preferred_element_type=jnp.float32)
            + jnp.dot(h[:, _KH2:], w2b_ref[0],
                      preferred_element_type=jnp.float32)
            + b2_ref[0])


def _ffn(x_slots, W1, b1, W2, b2, tile_expert, tile_valid):
    grid_spec = pltpu.PrefetchScalarGridSpec(
        num_scalar_prefetch=2,
        grid=(NT,),
        in_specs=[
            pl.BlockSpec((T, D_MODEL), lambda j, te, tv: (j, 0)),
            pl.BlockSpec((1, _KH1, D_FF), lambda j, te, tv: (te[j], 0, 0)),
            pl.BlockSpec((1, _KH1, D_FF), lambda j, te, tv: (te[j], 1, 0)),
            pl.BlockSpec((1, 1, D_FF), lambda j, te, tv: (te[j], 0, 0)),
            pl.BlockSpec((1, _KH2, D_MODEL), lambda j, te, tv: (te[j], 0, 0)),
            pl.BlockSpec((1, _KH2, D_MODEL), lambda j, te, tv: (te[j], 1, 0)),
            pl.BlockSpec((1, 1, D_MODEL), lambda j, te, tv: (te[j], 0, 0)),
        ],
        out_specs=pl.BlockSpec((T, D_MODEL), lambda j, te, tv: (j, 0)),
    )
    return pl.pallas_call(
        _ffn_body,
        grid_spec=grid_spec,
        out_shape=jax.ShapeDtypeStruct((NSLOT, D_MODEL), jnp.float32),
    )(tile_expert, tile_valid, x_slots,
      W1, W1, b1.reshape(E, 1, D_FF), W2, W2, b2.reshape(E, 1, D_MODEL))


def kernel(x, gate_W, gate_b, W1, b1, W2, b2):
    seq, bsz, dim = x.shape
    x_flat = x.reshape(seq * bsz, dim)
    slot, tile_expert, tile_valid = _gate(x_flat, gate_W, gate_b)
    x_slots = _make_scatter_rows(SEQ, NSLOT, D_MODEL)(x_flat, slot)
    y_slots = _ffn(x_slots, W1, b1, W2, b2, tile_expert, tile_valid)
    out = _make_gather_rows(SEQ, NSLOT, D_MODEL)(y_slots, slot)
    return out.reshape(seq, bsz, dim)


# speedup vs baseline: 1.0126x; 1.0126x over previous
import functools

import jax
import jax.numpy as jnp
from jax import lax
from jax.experimental import pallas as pl
from jax.experimental.pallas import tpu as pltpu
from jax.experimental.pallas import tpu_sc as plsc

SEQ = 2048
D_MODEL = 768
D_FF = 1024
E = 64
T = 64
NT = SEQ // T + E
NSLOT = NT * T


_TLOG = T.bit_length() - 1
assert (1 << _TLOG) == T


def _gate_body(x_ref, w_ref, b_ref, slot_ref, te_ref, tv_ref):
    scores = jnp.dot(x_ref[...], w_ref[...],
                     preferred_element_type=jnp.float32) + b_ref[...]
    m = jnp.max(scores, axis=1, keepdims=True)
    ii = lax.broadcasted_iota(jnp.int32, (SEQ, E), 1)
    eidx = jnp.min(jnp.where(scores == m, ii, E), axis=1, keepdims=True)
    onehot = (ii == eidx).astype(jnp.bfloat16)
    CKN, CK = 8, SEQ // 8
    rr = lax.broadcasted_iota(jnp.int32, (CK, CK), 0)
    cc = lax.broadcasted_iota(jnp.int32, (CK, CK), 1)
    ltri = (cc <= rr).astype(jnp.bfloat16)
    parts = [jnp.dot(ltri, onehot[k * CK:(k + 1) * CK, :],
                     preferred_element_type=jnp.float32) for k in range(CKN)]
    off = jnp.zeros((1, E), jnp.float32)
    fixed = []
    for k in range(CKN):
        fixed.append(parts[k] + off)
        off = off + parts[k][CK - 1:CK, :]
    csum = jnp.concatenate(fixed, axis=0)
    counts = csum[SEQ - 1:SEQ, :].astype(jnp.int32)
    onef = onehot.astype(jnp.float32)
    rank = (jnp.sum(onef * csum, axis=1, keepdims=True)
            - 1.0).astype(jnp.int32)
    tpe = (counts + (T - 1)) >> _TLOG
    ur = lax.broadcasted_iota(jnp.int32, (E, E), 0)
    uc = lax.broadcasted_iota(jnp.int32, (E, E), 1)
    utri = (ur <= uc).astype(jnp.bfloat16)
    ctiles = jnp.dot(tpe.astype(jnp.bfloat16), utri,
                     preferred_element_type=jnp.float32).astype(jnp.int32)
    tile_base = ctiles - tpe
    base_tok = jnp.sum(onef * tile_base.astype(jnp.float32),
                       axis=1, keepdims=True).astype(jnp.int32)
    slot_ref[...] = (base_tok + (rank >> _TLOG)) * T + (rank & (T - 1))
    jj = lax.broadcasted_iota(jnp.int32, (NT, E), 0)
    te = jnp.sum(jnp.where(ctiles <= jj, 1, 0), axis=1, keepdims=True)
    te_ref[...] = jnp.minimum(te, E - 1)
    total = ctiles[:, E - 1:E]
    jv = lax.broadcasted_iota(jnp.int32, (NT, 1), 0)
    tv_ref[...] = jnp.where(jv < total, 1, 0)


def _gate(x_flat, gate_W, gate_b):
    slot2, te2, tv2 = pl.pallas_call(
        _gate_body,
        out_shape=(
            jax.ShapeDtypeStruct((SEQ, 1), jnp.int32),
            jax.ShapeDtypeStruct((NT, 1), jnp.int32),
            jax.ShapeDtypeStruct((NT, 1), jnp.int32),
        ),
    )(x_flat, gate_W, gate_b.reshape(1, E))
    return slot2.reshape(SEQ), te2.reshape(NT), tv2.reshape(NT)


_NC, _NS = 2, 16
_NW = _NC * _NS


def _make_scatter_rows(n_rows, n_out, d):
    per_w = n_rows // _NW
    mesh = plsc.VectorSubcoreMesh(core_axis_name="c", subcore_axis_name="s")

    @functools.partial(
        pl.kernel, mesh=mesh,
        out_type=jax.ShapeDtypeStruct((n_out, d), jnp.float32),
        scratch_types=[
            pltpu.VMEM((per_w,), jnp.int32),
            pltpu.VMEM((per_w, d), jnp.float32),
            pltpu.SemaphoreType.DMA,
        ],
    )
    def k(table_hbm, idx_hbm, out_hbm, idx_v, rows_v, sem):
        wid = lax.axis_index("s") * _NC + lax.axis_index("c")
        base = wid * per_w
        pltpu.sync_copy(idx_hbm.at[pl.ds(base, per_w)], idx_v)
        pltpu.sync_copy(table_hbm.at[pl.ds(base, per_w)], rows_v)
        pltpu.async_copy(rows_v, out_hbm.at[idx_v], sem).wait()

    return k


def _make_gather_rows(n_rows, n_tab, d):
    per_w = n_rows // _NW
    mesh = plsc.VectorSubcoreMesh(core_axis_name="c", subcore_axis_name="s")

    @functools.partial(
        pl.kernel, mesh=mesh,
        out_type=jax.ShapeDtypeStruct((n_rows, d), jnp.float32),
        scratch_types=[
            pltpu.VMEM((per_w,), jnp.int32),
            pltpu.VMEM((per_w, d), jnp.float32),
            pltpu.SemaphoreType.DMA,
        ],
    )
    def k(table_hbm, idx_hbm, out_hbm, idx_v, rows_v, sem):
        wid = lax.axis_index("s") * _NC + lax.axis_index("c")
        base = wid * per_w
        pltpu.sync_copy(idx_hbm.at[pl.ds(base, per_w)], idx_v)
        pltpu.async_copy(table_hbm.at[idx_v], rows_v, sem).wait()
        pltpu.sync_copy(rows_v, out_hbm.at[pl.ds(base, per_w)])

    return k


def _ffn_body(te_ref, tv_ref, x_ref, w1_ref, b1_ref, w2_ref, b2_ref, o_ref):
    j = pl.program_id(0)

    @pl.when(tv_ref[j] == 1)
    def _():
        xt = x_ref[...]
        h = jnp.maximum(
            jnp.dot(xt, w1_ref[0], preferred_element_type=jnp.float32)
            + b1_ref[0], 0.0)
        o_ref[...] = (jnp.dot(h, w2_ref[0], preferred_element_type=jnp.float32)
                      + b2_ref[0])


def _ffn(x_slots, W1, b1, W2, b2, tile_expert, tile_valid):
    grid_spec = pltpu.PrefetchScalarGridSpec(
        num_scalar_prefetch=2,
        grid=(NT,),
        in_specs=[
            pl.BlockSpec((T, D_MODEL), lambda j, te, tv: (j, 0)),
            pl.BlockSpec((1, D_MODEL, D_FF), lambda j, te, tv: (te[j], 0, 0)),
            pl.BlockSpec((1, 1, D_FF), lambda j, te, tv: (te[j], 0, 0)),
            pl.BlockSpec((1, D_FF, D_MODEL), lambda j, te, tv: (te[j], 0, 0)),
            pl.BlockSpec((1, 1, D_MODEL), lambda j, te, tv: (te[j], 0, 0)),
        ],
        out_specs=pl.BlockSpec((T, D_MODEL), lambda j, te, tv: (j, 0)),
    )
    return pl.pallas_call(
        _ffn_body,
        grid_spec=grid_spec,
        out_shape=jax.ShapeDtypeStruct((NSLOT, D_MODEL), jnp.float32),
    )(tile_expert, tile_valid, x_slots,
      W1, b1.reshape(E, 1, D_FF), W2, b2.reshape(E, 1, D_MODEL))


def kernel(x, gate_W, gate_b, W1, b1, W2, b2):
    seq, bsz, dim = x.shape
    x_flat = x.reshape(seq * bsz, dim)
    slot, tile_expert, tile_valid = _gate(x_flat, gate_W, gate_b)
    x_slots = _make_scatter_rows(SEQ, NSLOT, D_MODEL)(x_flat, slot)
    y_slots = _ffn(x_slots, W1, b1, W2, b2, tile_expert, tile_valid)
    out = _make_gather_rows(SEQ, NSLOT, D_MODEL)(y_slots, slot)
    return out.reshape(seq, bsz, dim)
